# baseline (speedup 1.0000x reference)
import functools

import jax
import jax.numpy as jnp
from jax import lax
from jax.experimental import pallas as pl
from jax.experimental.pallas import tpu as pltpu
from jax.experimental.pallas import tpu_sc as plsc

_D = 32
_C = 8192
_N = 8192
_TN = 256
_NB = _N // _TN
_INT_MAX = jnp.iinfo(jnp.int32).max

_NUM_WORKERS = 32
_BW = _N // _NUM_WORKERS
_GCH = 128


_W = 2048
_NW_WIN = _C // _W


def _dist_argmin_body(x_ref, e_ref, z2_ref, e2_ref, idx_ref, acc_ref):
    i = pl.program_id(0)
    xb = (2.0 * x_ref[...]).astype(jnp.bfloat16)
    z2 = z2_ref[...]
    best_i = None
    best_v = None
    best_t = None
    for w in range(_NW_WIN):
        eb = e_ref[pl.ds(w * _W, _W), :].astype(jnp.bfloat16)
        mm = lax.dot_general(
            xb, eb,
            dimension_numbers=(((1,), (1,)), ((), ())),
            preferred_element_type=jnp.float32,
        )
        dist = (z2 - mm) + e2_ref[:, pl.ds(w * _W, _W)]
        minv = jnp.min(dist, axis=1, keepdims=True)
        iota = lax.broadcasted_iota(jnp.int32, (_TN, _W), 1) + w * _W
        idx = jnp.min(
            jnp.where(dist == minv, iota, _INT_MAX), axis=1, keepdims=True)
        minv_bf = minv.astype(jnp.bfloat16).astype(jnp.float32)
        if w == 0:
            best_i, best_v, best_t = idx, minv_bf, minv
        else:
            lt = minv < best_v
            best_i = jnp.where(lt, idx, best_i)
            best_t = jnp.where(lt, minv, best_t)
            best_v = jnp.where(lt, minv_bf, best_v)
    idx_ref[...] = best_i.reshape(1, 1, _TN)

    @pl.when(i == 0)
    def _():
        acc_ref[...] = jnp.zeros_like(acc_ref)

    acc_ref[...] = acc_ref[...] + jnp.sum(best_t).reshape(1, 1)

    @pl.when(i == pl.num_programs(0) - 1)
    def _():
        acc_ref[...] = acc_ref[...] * (1.0 / float(_N * _D))


def _tc_dist_argmin(x_flat, embed_weight, z2, e2_t):
    return pl.pallas_call(
        _dist_argmin_body,
        grid=(_NB,),
        in_specs=[
            pl.BlockSpec((_TN, _D), lambda i: (i, 0)),
            pl.BlockSpec((_C, _D), lambda i: (0, 0)),
            pl.BlockSpec((_TN, 1), lambda i: (i, 0)),
            pl.BlockSpec((1, _C), lambda i: (0, 0)),
        ],
        out_specs=[
            pl.BlockSpec((1, 1, _TN), lambda i: (i, 0, 0)),
            pl.BlockSpec((1, 1), lambda i: (0, 0)),
        ],
        out_shape=[
            jax.ShapeDtypeStruct((_NB, 1, _TN), jnp.int32),
            jax.ShapeDtypeStruct((1, 1), jnp.float32),
        ],
    )(x_flat, embed_weight, z2, e2_t)


def _sc_gather_body(e_hbm, idx_hbm, x_hbm, out_hbm, idx_v, rows_v, x_v, sem):
    wid = lax.axis_index("s") * 2 + lax.axis_index("c")
    base = wid * _BW
    pltpu.sync_copy(x_hbm.at[pl.ds(base, _BW)], x_v)
    for k in range(_BW // _GCH):
        pltpu.sync_copy(idx_hbm.at[pl.ds(base + k * _GCH, _GCH)], idx_v.at[k])
        pltpu.async_copy(
            e_hbm.at[idx_v.at[k]], rows_v.at[pl.ds(k * _GCH, _GCH)], sem
        ).wait()

    def body(t, carry):
        for c in range(_D // 16):
            sl = pl.ds(c * 16, 16)
            xv = x_v[t, sl]
            qv = rows_v[t, sl]
            rows_v[t, sl] = xv + (qv - xv)
        return carry

    lax.fori_loop(0, _BW, body, 0)
    pltpu.sync_copy(rows_v, out_hbm.at[pl.ds(base, _BW)])


@functools.cache
def _sc_gather():
    return pl.kernel(
        _sc_gather_body,
        out_type=jax.ShapeDtypeStruct((_N, _D), jnp.float32),
        mesh=plsc.VectorSubcoreMesh(core_axis_name="c", subcore_axis_name="s"),
        scratch_types=[
            pltpu.VMEM((_BW // _GCH, _GCH), jnp.int32),
            pltpu.VMEM((_BW, _D), jnp.float32),
            pltpu.VMEM((_BW, _D), jnp.float32),
            pltpu.SemaphoreType.DMA,
        ],
        compiler_params=pltpu.CompilerParams(use_tc_tiling_on_sc=False),
    )


def kernel(x, embed_weight):
    x_flat = x.reshape(-1, _D)
    z2 = jnp.sum(x_flat ** 2, axis=1, keepdims=True)
    e2_t = jnp.sum(embed_weight ** 2, axis=1, keepdims=True).T

    idx3, acc = _tc_dist_argmin(x_flat, embed_weight, z2, e2_t)
    ind_flat = idx3.reshape(-1)

    quantize_st = _sc_gather()(embed_weight, ind_flat, x_flat)

    diff = acc[0, 0]
    embed_ind = ind_flat.reshape(x.shape[:-1])
    return (quantize_st.reshape(x.shape), diff, embed_ind)


# speedup vs baseline: 1.0693x; 1.0693x over previous
import functools

import jax
import jax.numpy as jnp
from jax import lax
from jax.experimental import pallas as pl
from jax.experimental.pallas import tpu as pltpu
from jax.experimental.pallas import tpu_sc as plsc

_D = 32
_C = 8192
_N = 8192
_TN = 512
_NB = _N // _TN
_INT_MAX = jnp.iinfo(jnp.int32).max

_NUM_WORKERS = 32
_BW = _N // _NUM_WORKERS
_GCH = 128


_W = 2048
_NW_WIN = _C // _W


def _dist_argmin_body(x_ref, e_ref, z2_ref, e2_ref, idx_ref, acc_ref):
    i = pl.program_id(0)
    z2 = z2_ref[...]
    mm_full = lax.dot_general(
        x_ref[...], e_ref[...],
        dimension_numbers=(((1,), (1,)), ((), ())),
        preferred_element_type=jnp.float32,
    )
    best_i = None
    best_v = None
    best_t = None
    for w in range(_NW_WIN):
        dist = (z2 - mm_full[:, w * _W:(w + 1) * _W]) + e2_ref[:, pl.ds(w * _W, _W)]
        minv = jnp.min(dist, axis=1, keepdims=True)
        iota = lax.broadcasted_iota(jnp.int32, (_TN, _W), 1) + w * _W
        idx = jnp.min(
            jnp.where(dist == minv, iota, _INT_MAX), axis=1, keepdims=True)
        minv_bf = minv.astype(jnp.bfloat16).astype(jnp.float32)
        if w == 0:
            best_i, best_v, best_t = idx, minv_bf, minv
        else:
            lt = minv < best_v
            best_i = jnp.where(lt, idx, best_i)
            best_t = jnp.where(lt, minv, best_t)
            best_v = jnp.where(lt, minv_bf, best_v)
    idx_ref[...] = best_i.reshape(1, 1, _TN)

    @pl.when(i == 0)
    def _():
        acc_ref[...] = jnp.zeros_like(acc_ref)

    acc_ref[...] = acc_ref[...] + jnp.sum(best_t).reshape(1, 1)

    @pl.when(i == pl.num_programs(0) - 1)
    def _():
        acc_ref[...] = acc_ref[...] * (1.0 / float(_N * _D))


def _tc_dist_argmin(x_flat, embed_weight, z2, e2_t):
    return pl.pallas_call(
        _dist_argmin_body,
        grid=(_NB,),
        in_specs=[
            pl.BlockSpec((_TN, _D), lambda i: (i, 0)),
            pl.BlockSpec((_C, _D), lambda i: (0, 0)),
            pl.BlockSpec((_TN, 1), lambda i: (i, 0)),
            pl.BlockSpec((1, _C), lambda i: (0, 0)),
        ],
        out_specs=[
            pl.BlockSpec((1, 1, _TN), lambda i: (i, 0, 0)),
            pl.BlockSpec((1, 1), lambda i: (0, 0)),
        ],
        out_shape=[
            jax.ShapeDtypeStruct((_NB, 1, _TN), jnp.int32),
            jax.ShapeDtypeStruct((1, 1), jnp.float32),
        ],
    )(x_flat, embed_weight, z2, e2_t)


def _sc_gather_body(e_hbm, idx_hbm, x_hbm, out_hbm, idx_v, rows_v, x_v, sem):
    wid = lax.axis_index("s") * 2 + lax.axis_index("c")
    base = wid * _BW
    pltpu.sync_copy(x_hbm.at[pl.ds(base, _BW)], x_v)
    for k in range(_BW // _GCH):
        pltpu.sync_copy(idx_hbm.at[pl.ds(base + k * _GCH, _GCH)], idx_v.at[k])
        pltpu.async_copy(
            e_hbm.at[idx_v.at[k]], rows_v.at[pl.ds(k * _GCH, _GCH)], sem
        ).wait()

    def body(t, carry):
        for c in range(_D // 16):
            sl = pl.ds(c * 16, 16)
            xv = x_v[t, sl]
            qv = rows_v[t, sl]
            rows_v[t, sl] = xv + (qv - xv)
        return carry

    lax.fori_loop(0, _BW, body, 0)
    pltpu.sync_copy(rows_v, out_hbm.at[pl.ds(base, _BW)])


@functools.cache
def _sc_gather():
    return pl.kernel(
        _sc_gather_body,
        out_type=jax.ShapeDtypeStruct((_N, _D), jnp.float32),
        mesh=plsc.VectorSubcoreMesh(core_axis_name="c", subcore_axis_name="s"),
        scratch_types=[
            pltpu.VMEM((_BW // _GCH, _GCH), jnp.int32),
            pltpu.VMEM((_BW, _D), jnp.float32),
            pltpu.VMEM((_BW, _D), jnp.float32),
            pltpu.SemaphoreType.DMA,
        ],
        compiler_params=pltpu.CompilerParams(use_tc_tiling_on_sc=False),
    )


def kernel(x, embed_weight):
    x_flat = x.reshape(-1, _D)
    z2 = jnp.sum(x_flat ** 2, axis=1, keepdims=True)
    e2_t = jnp.sum(embed_weight ** 2, axis=1, keepdims=True).T
    xb2 = (2.0 * x_flat).astype(jnp.bfloat16)
    e_bf = embed_weight.astype(jnp.bfloat16)

    idx3, acc = _tc_dist_argmin(xb2, e_bf, z2, e2_t)
    ind_flat = idx3.reshape(-1)

    quantize_st = _sc_gather()(embed_weight, ind_flat, x_flat)

    diff = acc[0, 0]
    embed_ind = ind_flat.reshape(x.shape[:-1])
    return (quantize_st.reshape(x.shape), diff, embed_ind)


# speedup vs baseline: 1.0957x; 1.0247x over previous
import functools

import jax
import jax.numpy as jnp
from jax import lax
from jax.experimental import pallas as pl
from jax.experimental.pallas import tpu as pltpu
from jax.experimental.pallas import tpu_sc as plsc

_D = 32
_C = 8192
_N = 8192
_TN = 512
_NB = _N // _TN
_INT_MAX = jnp.iinfo(jnp.int32).max

_NUM_WORKERS = 32
_BW = _N // _NUM_WORKERS
_GCH = 128


_W = 2048
_NW_WIN = _C // _W


def _dist_argmin_body(x_ref, e_ref, z2_ref, e2_ref, idx_ref, acc_ref):
    i = pl.program_id(0)
    z2 = z2_ref[...]
    mm_full = lax.dot_general(
        x_ref[...], e_ref[...],
        dimension_numbers=(((1,), (1,)), ((), ())),
        preferred_element_type=jnp.float32,
    )
    best_i = None
    best_v = None
    best_t = None
    for w in range(_NW_WIN):
        dist = (z2 - mm_full[:, w * _W:(w + 1) * _W]) + e2_ref[:, pl.ds(w * _W, _W)]
        minv = jnp.min(dist, axis=1, keepdims=True)
        idx = jnp.argmin(dist, axis=1).astype(jnp.int32).reshape(_TN, 1) + w * _W
        minv_bf = minv.astype(jnp.bfloat16).astype(jnp.float32)
        if w == 0:
            best_i, best_v, best_t = idx, minv_bf, minv
        else:
            lt = minv < best_v
            best_i = jnp.where(lt, idx, best_i)
            best_t = jnp.where(lt, minv, best_t)
            best_v = jnp.where(lt, minv_bf, best_v)
    idx_ref[...] = best_i.reshape(1, 1, _TN)

    @pl.when(i == 0)
    def _():
        acc_ref[...] = jnp.zeros_like(acc_ref)

    acc_ref[...] = acc_ref[...] + jnp.sum(best_t).reshape(1, 1)

    @pl.when(i == pl.num_programs(0) - 1)
    def _():
        acc_ref[...] = acc_ref[...] * (1.0 / float(_N * _D))


def _tc_dist_argmin(x_flat, embed_weight, z2, e2_t):
    return pl.pallas_call(
        _dist_argmin_body,
        grid=(_NB,),
        in_specs=[
            pl.BlockSpec((_TN, _D), lambda i: (i, 0)),
            pl.BlockSpec((_C, _D), lambda i: (0, 0)),
            pl.BlockSpec((_TN, 1), lambda i: (i, 0)),
            pl.BlockSpec((1, _C), lambda i: (0, 0)),
        ],
        out_specs=[
            pl.BlockSpec((1, 1, _TN), lambda i: (i, 0, 0)),
            pl.BlockSpec((1, 1), lambda i: (0, 0)),
        ],
        out_shape=[
            jax.ShapeDtypeStruct((_NB, 1, _TN), jnp.int32),
            jax.ShapeDtypeStruct((1, 1), jnp.float32),
        ],
    )(x_flat, embed_weight, z2, e2_t)


def _sc_gather_body(e_hbm, idx_hbm, x_hbm, out_hbm, idx_v, rows_v, x_v, sem):
    wid = lax.axis_index("s") * 2 + lax.axis_index("c")
    base = wid * _BW
    pltpu.sync_copy(x_hbm.at[pl.ds(base, _BW)], x_v)
    for k in range(_BW // _GCH):
        pltpu.sync_copy(idx_hbm.at[pl.ds(base + k * _GCH, _GCH)], idx_v.at[k])
        pltpu.async_copy(
            e_hbm.at[idx_v.at[k]], rows_v.at[pl.ds(k * _GCH, _GCH)], sem
        ).wait()

    def body(t, carry):
        for c in range(_D // 16):
            sl = pl.ds(c * 16, 16)
            xv = x_v[t, sl]
            qv = rows_v[t, sl]
            rows_v[t, sl] = xv + (qv - xv)
        return carry

    lax.fori_loop(0, _BW, body, 0)
    pltpu.sync_copy(rows_v, out_hbm.at[pl.ds(base, _BW)])


@functools.cache
def _sc_gather():
    return pl.kernel(
        _sc_gather_body,
        out_type=jax.ShapeDtypeStruct((_N, _D), jnp.float32),
        mesh=plsc.VectorSubcoreMesh(core_axis_name="c", subcore_axis_name="s"),
        scratch_types=[
            pltpu.VMEM((_BW // _GCH, _GCH), jnp.int32),
            pltpu.VMEM((_BW, _D), jnp.float32),
            pltpu.VMEM((_BW, _D), jnp.float32),
            pltpu.SemaphoreType.DMA,
        ],
        compiler_params=pltpu.CompilerParams(use_tc_tiling_on_sc=False),
    )


def kernel(x, embed_weight):
    x_flat = x.reshape(-1, _D)
    z2 = jnp.sum(x_flat ** 2, axis=1, keepdims=True)
    e2_t = jnp.sum(embed_weight ** 2, axis=1, keepdims=True).T
    xb2 = (2.0 * x_flat).astype(jnp.bfloat16)
    e_bf = embed_weight.astype(jnp.bfloat16)

    idx3, acc = _tc_dist_argmin(xb2, e_bf, z2, e2_t)
    ind_flat = idx3.reshape(-1)

    quantize_st = _sc_gather()(embed_weight, ind_flat, x_flat)

    diff = acc[0, 0]
    embed_ind = ind_flat.reshape(x.shape[:-1])
    return (quantize_st.reshape(x.shape), diff, embed_ind)


# speedup vs baseline: 1.1344x; 1.0353x over previous
import functools

import jax
import jax.numpy as jnp
from jax import lax
from jax.experimental import pallas as pl
from jax.experimental.pallas import tpu as pltpu
from jax.experimental.pallas import tpu_sc as plsc

_D = 32
_C = 8192
_N = 8192
_TN = 512
_NB = _N // _TN
_INT_MAX = jnp.iinfo(jnp.int32).max

_NUM_WORKERS = 32
_BW = _N // _NUM_WORKERS
_GCH = 128


_W = 2048
_NW_WIN = _C // _W


def _dist_argmin_body(x_ref, e_ref, z2_ref, e2_ref, idx_ref, acc_ref):
    i = pl.program_id(0)
    z2 = z2_ref[...]
    mm_full = lax.dot_general(
        e_ref[...], x_ref[...],
        dimension_numbers=(((1,), (1,)), ((), ())),
        preferred_element_type=jnp.float32,
    )
    iota0 = lax.broadcasted_iota(jnp.int32, (_W, _TN), 0)
    best_i = None
    best_v = None
    best_t = None
    for w in range(_NW_WIN):
        dist = (z2 - mm_full[w * _W:(w + 1) * _W, :]) + e2_ref[pl.ds(w * _W, _W), :]
        minv = jnp.min(dist, axis=0, keepdims=True)
        idx = jnp.min(jnp.where(dist == minv, iota0, _INT_MAX),
                      axis=0, keepdims=True) + w * _W
        minv_bf = minv.astype(jnp.bfloat16).astype(jnp.float32)
        if w == 0:
            best_i, best_v, best_t = idx, minv_bf, minv
        else:
            lt = minv < best_v
            best_i = jnp.where(lt, idx, best_i)
            best_t = jnp.where(lt, minv, best_t)
            best_v = jnp.where(lt, minv_bf, best_v)
    idx_ref[...] = best_i.reshape(1, 1, _TN)

    @pl.when(i == 0)
    def _():
        acc_ref[...] = jnp.zeros_like(acc_ref)

    acc_ref[...] = acc_ref[...] + jnp.sum(best_t).reshape(1, 1)

    @pl.when(i == pl.num_programs(0) - 1)
    def _():
        acc_ref[...] = acc_ref[...] * (1.0 / float(_N * _D))


def _tc_dist_argmin(x_flat, embed_weight, z2, e2_t):
    return pl.pallas_call(
        _dist_argmin_body,
        grid=(_NB,),
        in_specs=[
            pl.BlockSpec((_TN, _D), lambda i: (i, 0)),
            pl.BlockSpec((_C, _D), lambda i: (0, 0)),
            pl.BlockSpec((1, _TN), lambda i: (0, i)),
            pl.BlockSpec((_C, 1), lambda i: (0, 0)),
        ],
        out_specs=[
            pl.BlockSpec((1, 1, _TN), lambda i: (i, 0, 0)),
            pl.BlockSpec((1, 1), lambda i: (0, 0)),
        ],
        out_shape=[
            jax.ShapeDtypeStruct((_NB, 1, _TN), jnp.int32),
            jax.ShapeDtypeStruct((1, 1), jnp.float32),
        ],
    )(x_flat, embed_weight, z2, e2_t)


def _sc_gather_body(e_hbm, idx_hbm, x_hbm, out_hbm, idx_v, rows_v, x_v, sem):
    wid = lax.axis_index("s") * 2 + lax.axis_index("c")
    base = wid * _BW
    pltpu.sync_copy(x_hbm.at[pl.ds(base, _BW)], x_v)
    for k in range(_BW // _GCH):
        pltpu.sync_copy(idx_hbm.at[pl.ds(base + k * _GCH, _GCH)], idx_v.at[k])
        pltpu.async_copy(
            e_hbm.at[idx_v.at[k]], rows_v.at[pl.ds(k * _GCH, _GCH)], sem
        ).wait()

    def body(t, carry):
        for c in range(_D // 16):
            sl = pl.ds(c * 16, 16)
            xv = x_v[t, sl]
            qv = rows_v[t, sl]
            rows_v[t, sl] = xv + (qv - xv)
        return carry

    lax.fori_loop(0, _BW, body, 0)
    pltpu.sync_copy(rows_v, out_hbm.at[pl.ds(base, _BW)])


@functools.cache
def _sc_gather():
    return pl.kernel(
        _sc_gather_body,
        out_type=jax.ShapeDtypeStruct((_N, _D), jnp.float32),
        mesh=plsc.VectorSubcoreMesh(core_axis_name="c", subcore_axis_name="s"),
        scratch_types=[
            pltpu.VMEM((_BW // _GCH, _GCH), jnp.int32),
            pltpu.VMEM((_BW, _D), jnp.float32),
            pltpu.VMEM((_BW, _D), jnp.float32),
            pltpu.SemaphoreType.DMA,
        ],
        compiler_params=pltpu.CompilerParams(use_tc_tiling_on_sc=False),
    )


def kernel(x, embed_weight):
    x_flat = x.reshape(-1, _D)
    z2_t = jnp.sum(x_flat ** 2, axis=1, keepdims=True).T
    e2_c = jnp.sum(embed_weight ** 2, axis=1, keepdims=True)
    xb2 = (2.0 * x_flat).astype(jnp.bfloat16)
    e_bf = embed_weight.astype(jnp.bfloat16)

    idx3, acc = _tc_dist_argmin(xb2, e_bf, z2_t, e2_c)
    ind_flat = idx3.reshape(-1)

    quantize_st = _sc_gather()(embed_weight, ind_flat, x_flat)

    diff = acc[0, 0]
    embed_ind = ind_flat.reshape(x.shape[:-1])
    return (quantize_st.reshape(x.shape), diff, embed_ind)


# speedup vs baseline: 1.1735x; 1.0345x over previous
import functools

import jax
import jax.numpy as jnp
from jax import lax
from jax.experimental import pallas as pl
from jax.experimental.pallas import tpu as pltpu
from jax.experimental.pallas import tpu_sc as plsc

_D = 32
_C = 8192
_N = 8192
_TN = 1024
_NB = _N // _TN
_INT_MAX = jnp.iinfo(jnp.int32).max

_NUM_WORKERS = 32
_BW = _N // _NUM_WORKERS
_GCH = 128


_W = 2048
_NW_WIN = _C // _W


def _dist_argmin_body(x_ref, e_ref, z2_ref, e2_ref, idx_ref, acc_ref):
    i = pl.program_id(0)
    z2 = z2_ref[...]
    mm_full = lax.dot_general(
        e_ref[...], x_ref[...],
        dimension_numbers=(((1,), (1,)), ((), ())),
        preferred_element_type=jnp.float32,
    )
    iota0 = lax.broadcasted_iota(jnp.int32, (_W, _TN), 0)
    best_i = None
    best_v = None
    best_t = None
    for w in range(_NW_WIN):
        dist = (z2 - mm_full[w * _W:(w + 1) * _W, :]) + e2_ref[pl.ds(w * _W, _W), :]
        minv = jnp.min(dist, axis=0, keepdims=True)
        idx = jnp.min(jnp.where(dist == minv, iota0, _INT_MAX),
                      axis=0, keepdims=True) + w * _W
        minv_bf = minv.astype(jnp.bfloat16).astype(jnp.float32)
        if w == 0:
            best_i, best_v, best_t = idx, minv_bf, minv
        else:
            lt = minv < best_v
            best_i = jnp.where(lt, idx, best_i)
            best_t = jnp.where(lt, minv, best_t)
            best_v = jnp.where(lt, minv_bf, best_v)
    idx_ref[...] = best_i.reshape(1, 1, _TN)

    @pl.when(i == 0)
    def _():
        acc_ref[...] = jnp.zeros_like(acc_ref)

    acc_ref[...] = acc_ref[...] + jnp.sum(best_t).reshape(1, 1)

    @pl.when(i == pl.num_programs(0) - 1)
    def _():
        acc_ref[...] = acc_ref[...] * (1.0 / float(_N * _D))


def _tc_dist_argmin(x_flat, embed_weight, z2, e2_t):
    return pl.pallas_call(
        _dist_argmin_body,
        grid=(_NB,),
        in_specs=[
            pl.BlockSpec((_TN, _D), lambda i: (i, 0)),
            pl.BlockSpec((_C, _D), lambda i: (0, 0)),
            pl.BlockSpec((1, _TN), lambda i: (0, i)),
            pl.BlockSpec((_C, 1), lambda i: (0, 0)),
        ],
        out_specs=[
            pl.BlockSpec((1, 1, _TN), lambda i: (i, 0, 0)),
            pl.BlockSpec((1, 1), lambda i: (0, 0)),
        ],
        out_shape=[
            jax.ShapeDtypeStruct((_NB, 1, _TN), jnp.int32),
            jax.ShapeDtypeStruct((1, 1), jnp.float32),
        ],
    )(x_flat, embed_weight, z2, e2_t)


def _sc_gather_body(e_hbm, idx_hbm, x_hbm, out_hbm, idx_v, rows_v, x_v, sem):
    wid = lax.axis_index("s") * 2 + lax.axis_index("c")
    base = wid * _BW
    pltpu.sync_copy(x_hbm.at[pl.ds(base, _BW)], x_v)
    for k in range(_BW // _GCH):
        pltpu.sync_copy(idx_hbm.at[pl.ds(base + k * _GCH, _GCH)], idx_v.at[k])
        pltpu.async_copy(
            e_hbm.at[idx_v.at[k]], rows_v.at[pl.ds(k * _GCH, _GCH)], sem
        ).wait()

    def body(t, carry):
        for c in range(_D // 16):
            sl = pl.ds(c * 16, 16)
            xv = x_v[t, sl]
            qv = rows_v[t, sl]
            rows_v[t, sl] = xv + (qv - xv)
        return carry

    lax.fori_loop(0, _BW, body, 0)
    pltpu.sync_copy(rows_v, out_hbm.at[pl.ds(base, _BW)])


@functools.cache
def _sc_gather():
    return pl.kernel(
        _sc_gather_body,
        out_type=jax.ShapeDtypeStruct((_N, _D), jnp.float32),
        mesh=plsc.VectorSubcoreMesh(core_axis_name="c", subcore_axis_name="s"),
        scratch_types=[
            pltpu.VMEM((_BW // _GCH, _GCH), jnp.int32),
            pltpu.VMEM((_BW, _D), jnp.float32),
            pltpu.VMEM((_BW, _D), jnp.float32),
            pltpu.SemaphoreType.DMA,
        ],
        compiler_params=pltpu.CompilerParams(use_tc_tiling_on_sc=False),
    )


def kernel(x, embed_weight):
    x_flat = x.reshape(-1, _D)
    z2_t = jnp.sum(x_flat ** 2, axis=1, keepdims=True).T
    e2_c = jnp.sum(embed_weight ** 2, axis=1, keepdims=True)
    xb2 = (2.0 * x_flat).astype(jnp.bfloat16)
    e_bf = embed_weight.astype(jnp.bfloat16)

    idx3, acc = _tc_dist_argmin(xb2, e_bf, z2_t, e2_c)
    ind_flat = idx3.reshape(-1)

    quantize_st = _sc_gather()(embed_weight, ind_flat, x_flat)

    diff = acc[0, 0]
    embed_ind = ind_flat.reshape(x.shape[:-1])
    return (quantize_st.reshape(x.shape), diff, embed_ind)
